# Initial kernel scaffold; baseline (speedup 1.0000x reference)
import jax
import jax.numpy as jnp
from jax.experimental import pallas as pl


def kernel(x, pos, edge_index, params):
    raise NotImplementedError("write your pallas kernel here")


# speedup vs baseline: 1.8441x; 1.8441x over previous
import functools

import jax
import jax.numpy as jnp
from jax import lax
from jax.experimental import pallas as pl
from jax.experimental.pallas import tpu as pltpu
from jax.experimental.pallas import tpu_sc as plsc

N = 10000
E = 320000
D_IN = 128
H = 64

NC, NS, LANES = 2, 16, 16
NW = NC * NS
ROWS_PER_TILE = -(-N // (NW * 8)) * 8
N_PAD = NW * ROWS_PER_TILE
WB = 80
E_PER_W = E // NW
NWIN_B1 = E_PER_W // WB

NEG_INF = float("-inf")


def _mesh():
    return plsc.VectorSubcoreMesh(
        core_axis_name="c", subcore_axis_name="s",
        num_cores=NC, num_subcores=NS)


def _wid():
    return lax.axis_index("s") * NC + lax.axis_index("c")


@functools.cache
def _build_b1():
  return pl.kernel(
    _b1_body,
    out_type=jax.ShapeDtypeStruct((E, H), jnp.float32),
    mesh=_mesh(),
    scratch_types=[
        pltpu.VMEM((WB,), jnp.int32),
        pltpu.VMEM((WB,), jnp.int32),
        pltpu.VMEM((WB, 2 * H), jnp.float32),
        pltpu.VMEM((WB, 2 * H), jnp.float32),
        pltpu.VMEM((WB, H), jnp.float32),
        pltpu.SemaphoreType.DMA,
        pltpu.SemaphoreType.DMA,
    ],
  )


def _b1_body(st_hbm, src_hbm, dst_hbm, hid_hbm,
             idx_s, idx_t, rows_s, rows_t, hid_buf, sem_s, sem_t):
    base = _wid() * E_PER_W

    def win(w, _):
        ws = base + w * WB
        pltpu.sync_copy(src_hbm.at[pl.ds(ws, WB)], idx_s)
        pltpu.sync_copy(dst_hbm.at[pl.ds(ws, WB)], idx_t)
        cs = pltpu.async_copy(st_hbm.at[idx_s], rows_s, sem_s)
        ct = pltpu.async_copy(st_hbm.at[idx_t], rows_t, sem_t)
        cs.wait()
        ct.wait()

        def row(i, _):
            for v in range(H // LANES):
                sl = pl.ds(v * LANES, LANES)
                tl = pl.ds(H + v * LANES, LANES)
                hid_buf[i, sl] = jnp.maximum(rows_s[i, sl] + rows_t[i, tl],
                                             0.0)
            return 0

        lax.fori_loop(0, WB, row, 0)
        pltpu.sync_copy(hid_buf, hid_hbm.at[pl.ds(ws, WB), :])
        return 0

    lax.fori_loop(0, NWIN_B1, win, 0)


@functools.cache
def _build_b3():
  return pl.kernel(
    _b3_body,
    out_type=jax.ShapeDtypeStruct((N_PAD, H), jnp.float32),
    mesh=_mesh(),
    scratch_types=[
        pltpu.VMEM((ROWS_PER_TILE, H), jnp.float32),
        pltpu.VMEM((WB, H), jnp.float32),
        pltpu.VMEM((WB,), jnp.int32),
        pltpu.VMEM((NW * LANES,), jnp.int32),
    ],
  )


def _b3_body(e_hbm, dst_hbm, bounds_hbm, agg_hbm, acc, e_buf, dst_buf, bnd):
    wid = _wid()
    base = wid * ROWS_PER_TILE
    row_count = jnp.minimum(ROWS_PER_TILE, N - base)

    pltpu.sync_copy(bounds_hbm, bnd)

    def initrow(i, _):
        for v in range(H // LANES):
            acc[i, pl.ds(v * LANES, LANES)] = jnp.full((LANES,), NEG_INF,
                                                       jnp.float32)
        return 0

    lax.fori_loop(0, ROWS_PER_TILE, initrow, 0)

    bv = bnd[pl.ds(wid * LANES, LANES)]
    start = bv[0]
    end = bv[1]
    s8 = start // 8
    astart = s8 * 8
    cap8 = (E - WB) // 8
    n_win = jnp.maximum((end - astart + WB - 1) // WB, 0)

    def win(w, _):
        ws = jnp.minimum(s8 + w * (WB // 8), cap8) * 8
        pltpu.sync_copy(dst_hbm.at[pl.ds(ws, WB)], dst_buf)
        pltpu.sync_copy(e_hbm.at[pl.ds(ws, WB), :], e_buf)

        def grp(j, _):
            dvec = dst_buf[pl.ds(j * LANES, LANES)] - base
            for k in range(LANES):
                d = dvec[k]
                i = j * LANES + k

                @pl.when(jnp.logical_and(d >= 0, d < row_count))
                def _(d=d, i=i):
                    for v in range(H // LANES):
                        sl = pl.ds(v * LANES, LANES)
                        acc[d, sl] = jnp.maximum(acc[d, sl], e_buf[i, sl])

            return 0

        lax.fori_loop(0, WB // LANES, grp, 0)
        return 0

    lax.fori_loop(0, n_win, win, 0)
    pltpu.sync_copy(acc, agg_hbm.at[pl.ds(base, ROWS_PER_TILE), :])


def _b1_gather_relu(st, ssrc, sdst):
    return _build_b1()(st, ssrc, sdst)


def _b3_segmax(e, sdst, bounds):
    return _build_b3()(e, sdst, bounds)


BLK = 1000
GRID = N // BLK
EBLK = 2000
EGRID = E // EBLK

_full = lambda shape: pl.BlockSpec(shape, lambda i: (0, 0))
_rows = lambda w: pl.BlockSpec((BLK, w), lambda i: (i, 0))


def _dot(a, b):
    return jnp.dot(a, b, preferred_element_type=jnp.float32)


def _st_tables(z, pos_blk, wh1, bh1, wh2, bh2, a3, w1x, bf1):
    p = _dot(pos_blk, a3)
    delta = _dot(jnp.maximum(_dot(z, wh1) + bh1, 0.0), wh2) + bh2
    t = _dot(delta, a3) - p
    s = _dot(z, w1x) + p + bf1
    return jnp.concatenate([s, t], axis=-1)


def _pre_body(x_ref, pos_ref, we1, be1, we2, be2, wt, bt,
              wh1, bh1, wh2, bh2, a3, w1x, bf1,
              z_ref, st_ref):
    h = jnp.maximum(_dot(x_ref[...], we1[...]) + be1[...], 0.0)
    z = _dot(_dot(h, we2[...]) + be2[...], wt[...]) + bt[...]
    z_ref[...] = z
    st_ref[...] = _st_tables(z, pos_ref[...], wh1[...], bh1[...], wh2[...],
                             bh2[...], a3[...], w1x[...], bf1[...])


def _gmlp(agg_ref, z, wg1, bg1, wg2, bg2, final_relu):
    a = agg_ref[...]
    a = jnp.where(jnp.isfinite(a), a, 0.0)
    u = jnp.maximum(_dot(a, wg1[...]) + bg1[...], 0.0)
    out = _dot(u, wg2[...]) + bg2[...] + z
    if final_relu:
        out = jnp.maximum(out, 0.0)
    return out


def _mid_body(agg_ref, z_ref, pos_ref, wg1, bg1, wg2, bg2,
              wh1, bh1, wh2, bh2, a3, w1x, bf1,
              z_out, st_ref):
    z = _gmlp(agg_ref, z_ref[...], wg1, bg1, wg2, bg2, True)
    z_out[...] = z
    st_ref[...] = _st_tables(z, pos_ref[...], wh1[...], bh1[...], wh2[...],
                             bh2[...], a3[...], w1x[...], bf1[...])


def _mid3_body(agg_ref, z_ref, pos_ref, wg1, bg1, wg2, bg2, wp, bp,
               wh1, bh1, wh2, bh2, a3, w1x, bf1,
               z_out, st_ref):
    z = _gmlp(agg_ref, z_ref[...], wg1, bg1, wg2, bg2, True)
    w = _dot(z, wp[...]) + bp[...]
    z_out[...] = w
    st_ref[...] = _st_tables(w, pos_ref[...], wh1[...], bh1[...], wh2[...],
                             bh2[...], a3[...], w1x[...], bf1[...])


def _post_body(agg_ref, z_ref, wg1, bg1, wg2, bg2, wd1, bd1, wd2, bd2,
               out_ref):
    y = _gmlp(agg_ref, z_ref[...], wg1, bg1, wg2, bg2, False)
    h = jnp.maximum(_dot(y, wd1[...]) + bd1[...], 0.0)
    out_ref[...] = _dot(h, wd2[...]) + bd2[...]


def _b2_body(hid_ref, w2, b2, e_ref):
    e_ref[...] = _dot(hid_ref[...], w2[...]) + b2[...]


def _nodes_out(k):
    return [jax.ShapeDtypeStruct((N, k), jnp.float32)]


_W64 = _full((H, H))
_B64 = _full((1, H))
_A3 = _full((3, H))
_W3 = _full((H, 3))
_B3S = _full((1, 3))

_ST_WSPECS = [_W64, _B64, _W3, _B3S, _A3, _W64, _B64]

_pre_call = pl.pallas_call(
    _pre_body,
    grid=(GRID,),
    in_specs=[_rows(D_IN), _rows(3),
              _full((D_IN, D_IN)), _full((1, D_IN)), _full((D_IN, H)), _B64,
              _W64, _B64] + _ST_WSPECS,
    out_specs=[_rows(H), _rows(2 * H)],
    out_shape=[jax.ShapeDtypeStruct((N, H), jnp.float32),
               jax.ShapeDtypeStruct((N, 2 * H), jnp.float32)],
)

_mid_call = pl.pallas_call(
    _mid_body,
    grid=(GRID,),
    in_specs=[_rows(H), _rows(H), _rows(3),
              _W64, _B64, _W64, _B64] + _ST_WSPECS,
    out_specs=[_rows(H), _rows(2 * H)],
    out_shape=[jax.ShapeDtypeStruct((N, H), jnp.float32),
               jax.ShapeDtypeStruct((N, 2 * H), jnp.float32)],
)

_mid3_call = pl.pallas_call(
    _mid3_body,
    grid=(GRID,),
    in_specs=[_rows(H), _rows(H), _rows(3),
              _W64, _B64, _W64, _B64, _W64, _B64] + _ST_WSPECS,
    out_specs=[_rows(H), _rows(2 * H)],
    out_shape=[jax.ShapeDtypeStruct((N, H), jnp.float32),
               jax.ShapeDtypeStruct((N, 2 * H), jnp.float32)],
)

_post_call = pl.pallas_call(
    _post_body,
    grid=(GRID,),
    in_specs=[_rows(H), _rows(H),
              _W64, _B64, _W64, _B64, _W64, _B64, _full((H, 4)), _full((1, 4))],
    out_specs=pl.BlockSpec((BLK, 4), lambda i: (i, 0)),
    out_shape=jax.ShapeDtypeStruct((N, 4), jnp.float32),
)

_b2_call = pl.pallas_call(
    _b2_body,
    grid=(EGRID,),
    in_specs=[pl.BlockSpec((EBLK, H), lambda i: (i, 0)), _W64, _B64],
    out_specs=pl.BlockSpec((EBLK, H), lambda i: (i, 0)),
    out_shape=jax.ShapeDtypeStruct((E, H), jnp.float32),
)


def _conv_weights(p):
    (wh1, bh1), (wh2, bh2) = p['h']
    (wf1, bf1), (wf2, bf2) = p['f']
    a3 = wf1[:3]
    w1x = wf1[3:]
    return (wh1, bh1.reshape(1, H), wh2, bh2.reshape(1, 3), a3, w1x,
            bf1.reshape(1, H)), (wf2, bf2.reshape(1, H))


def _g_weights(p):
    (wg1, bg1), (wg2, bg2) = p['g']
    return wg1, bg1.reshape(1, H), wg2, bg2.reshape(1, H)


def kernel(x, pos, edge_index, params):
    src = edge_index[0]
    dst = edge_index[1]
    sdst, ssrc = lax.sort((dst, src), num_keys=1)
    tile_edges = jnp.arange(0, (NW + 1) * ROWS_PER_TILE, ROWS_PER_TILE,
                            dtype=jnp.int32)
    cuts = jnp.searchsorted(sdst, tile_edges).astype(jnp.int32)
    se = jnp.stack([cuts[:-1], cuts[1:]], axis=1)
    bounds = jnp.pad(se, ((0, 0), (0, LANES - 2))).reshape(NW * LANES)

    (we1, be1), (we2, be2) = params['encoder']
    ((wt, bt),) = params['input_transform']

    st_w, f2_w = _conv_weights(params['convs'][0])
    z, st = _pre_call(x, pos, we1, be1.reshape(1, D_IN), we2,
                      be2.reshape(1, H), wt, bt.reshape(1, H), *st_w)

    convs = list(params['convs']) + [params['out_conv']]
    for k in range(4):
        wf2, bf2 = f2_w
        hid = _b1_gather_relu(st, ssrc, sdst)
        e = _b2_call(hid, wf2, bf2)
        aggp = _b3_segmax(e, sdst, bounds)
        agg = aggp[:N]
        gw = _g_weights(convs[k])
        if k < 2:
            st_w, f2_w = _conv_weights(convs[k + 1])
            z, st = _mid_call(agg, z, pos, *gw, *st_w)
        elif k == 2:
            ((wp, bp),) = params['out_projection']
            st_w, f2_w = _conv_weights(convs[3])
            z, st = _mid3_call(agg, z, pos, *gw, wp, bp.reshape(1, H),
                               *st_w)
        else:
            (wd1, bd1), (wd2, bd2) = params['decoder']
            out = _post_call(agg, z, *gw, wd1, bd1.reshape(1, H),
                             wd2, bd2.reshape(1, 4))
    return out


# speedup vs baseline: 2.9549x; 1.6024x over previous
import functools

import jax
import jax.numpy as jnp
from jax import lax
from jax.experimental import pallas as pl
from jax.experimental.pallas import tpu as pltpu
from jax.experimental.pallas import tpu_sc as plsc

N = 10000
E = 320000
D_IN = 128
H = 64

NC, NS, LANES = 2, 16, 16
NW = NC * NS
ROWS_PER_TILE = -(-N // (NW * 8)) * 8
N_PAD = NW * ROWS_PER_TILE
WB = 80
E_PER_W = E // NW
NWIN_B1 = E_PER_W // WB

NEG_INF = float("-inf")


def _mesh():
    return plsc.VectorSubcoreMesh(
        core_axis_name="c", subcore_axis_name="s",
        num_cores=NC, num_subcores=NS)


def _wid():
    return lax.axis_index("s") * NC + lax.axis_index("c")


@functools.cache
def _build_b1():
  return pl.kernel(
    _b1_body,
    out_type=jax.ShapeDtypeStruct((E, H), jnp.float32),
    mesh=_mesh(),
    scratch_types=[
        pltpu.VMEM((2, WB), jnp.int32),
        pltpu.VMEM((2, WB), jnp.int32),
        pltpu.VMEM((2, WB, 2 * H), jnp.float32),
        pltpu.VMEM((2, WB, 2 * H), jnp.float32),
        pltpu.VMEM((2, WB, H), jnp.float32),
        pltpu.SemaphoreType.DMA,
        pltpu.SemaphoreType.DMA,
        pltpu.SemaphoreType.DMA,
        pltpu.SemaphoreType.DMA,
        pltpu.SemaphoreType.DMA,
        pltpu.SemaphoreType.DMA,
    ],
  )


def _b1_body(st_hbm, src_hbm, dst_hbm, hid_hbm,
             idx_s, idx_t, rows_s, rows_t, hid_buf,
             si0, si1, sg0, sg1, sw0, sw1):
    base = _wid() * E_PER_W
    sem_i = (si0, si1)
    sem_g = (sg0, sg1)
    sem_w = (sw0, sw1)
    NWIN = NWIN_B1

    def idx_copies(w, b):
        ws = base + w * WB
        return (pltpu.make_async_copy(src_hbm.at[pl.ds(ws, WB)],
                                      idx_s.at[b], sem_i[b]),
                pltpu.make_async_copy(dst_hbm.at[pl.ds(ws, WB)],
                                      idx_t.at[b], sem_i[b]))

    def gather_copies(b):
        return (pltpu.make_async_copy(st_hbm.at[idx_s.at[b]],
                                      rows_s.at[b], sem_g[b]),
                pltpu.make_async_copy(st_hbm.at[idx_t.at[b]],
                                      rows_t.at[b], sem_g[b]))

    def wb_copy(w, b):
        ws = base + w * WB
        return pltpu.make_async_copy(hid_buf.at[b],
                                     hid_hbm.at[pl.ds(ws, WB), :], sem_w[b])

    for c in idx_copies(0, 0):
        c.start()
    for c in idx_copies(0, 0):
        c.wait()
    for c in gather_copies(0):
        c.start()
    for c in idx_copies(1, 1):
        c.start()

    def step(w, b):
        nb = 1 - b

        @pl.when(w + 1 < NWIN)
        def _():
            for c in idx_copies(w + 1, nb):
                c.wait()
            for c in gather_copies(nb):
                c.start()

        @pl.when(w < NWIN)
        def _():
            for c in gather_copies(b):
                c.wait()

        @pl.when(w + 2 < NWIN)
        def _():
            for c in idx_copies(w + 2, b):
                c.start()

        @pl.when(jnp.logical_and(w >= 2, w - 2 < NWIN))
        def _():
            wb_copy(w - 2, b).wait()

        for i in range(WB):
            for v in range(H // LANES):
                sl = pl.ds(v * LANES, LANES)
                tl = pl.ds(H + v * LANES, LANES)
                hid_buf[b, i, sl] = jnp.maximum(
                    rows_s[b, i, sl] + rows_t[b, i, tl], 0.0)

        @pl.when(w < NWIN)
        def _():
            wb_copy(w, b).start()

    def pair(wp, _):
        for b in (0, 1):
            step(2 * wp + b, b)
        return 0

    lax.fori_loop(0, (NWIN + 1) // 2, pair, 0)
    wb_copy(NWIN - 1, (NWIN - 1) % 2).wait()


@functools.cache
def _build_b3():
  return pl.kernel(
    _b3_body,
    out_type=jax.ShapeDtypeStruct((N_PAD, H), jnp.float32),
    mesh=_mesh(),
    scratch_types=[
        pltpu.VMEM((ROWS_PER_TILE, H), jnp.float32),
        pltpu.VMEM((2, WB, H), jnp.float32),
        pltpu.VMEM((2, WB), jnp.int32),
        pltpu.VMEM((NW * LANES,), jnp.int32),
        pltpu.SemaphoreType.DMA,
        pltpu.SemaphoreType.DMA,
        pltpu.SemaphoreType.DMA,
        pltpu.SemaphoreType.DMA,
    ],
  )


def _b3_body(e_hbm, dst_hbm, bounds_hbm, agg_hbm, acc, e_buf, dst_buf, bnd,
             se0, se1, sd0, sd1):
    wid = _wid()
    base = wid * ROWS_PER_TILE
    row_count = jnp.minimum(ROWS_PER_TILE, N - base)
    sem_e = (se0, se1)
    sem_d = (sd0, sd1)

    pltpu.sync_copy(bounds_hbm, bnd)

    def initrow(i, _):
        for v in range(H // LANES):
            acc[i, pl.ds(v * LANES, LANES)] = jnp.full((LANES,), NEG_INF,
                                                       jnp.float32)
        return 0

    lax.fori_loop(0, ROWS_PER_TILE, initrow, 0)

    bv = bnd[pl.ds(wid * LANES, LANES)]
    start = bv[0]
    end = bv[1]
    s8 = start // 8
    astart = s8 * 8
    cap8 = (E - WB) // 8
    n_win = jnp.maximum((end - astart + WB - 1) // WB, 0)

    def _ws(w):
        return jnp.minimum(s8 + w * (WB // 8), cap8) * 8

    def _copies(w, b):
        ws = _ws(w)
        return (pltpu.make_async_copy(e_hbm.at[pl.ds(ws, WB), :],
                                      e_buf.at[b], sem_e[b]),
                pltpu.make_async_copy(dst_hbm.at[pl.ds(ws, WB)],
                                      dst_buf.at[b], sem_d[b]))

    @pl.when(n_win > 0)
    def _():
        for c in _copies(0, 0):
            c.start()

    def _window(w, par):
        def grp(j, _):
            dvec = dst_buf[par, pl.ds(j * LANES, LANES)] - base
            d0 = dvec[0]
            dlast = dvec[LANES - 1]
            uniform = d0 == dlast
            valid0 = jnp.logical_and(d0 >= 0, d0 < row_count)

            @pl.when(jnp.logical_and(uniform, valid0))
            def _():
                for v in range(H // LANES):
                    sl = pl.ds(v * LANES, LANES)
                    m = e_buf[par, j * LANES, sl]
                    for k in range(1, LANES):
                        m = jnp.maximum(m, e_buf[par, j * LANES + k, sl])
                    acc[d0, sl] = jnp.maximum(acc[d0, sl], m)

            @pl.when(jnp.logical_not(uniform))
            def _():
                c = (d0,) + tuple(
                    e_buf[par, j * LANES, pl.ds(v * LANES, LANES)]
                    for v in range(H // LANES))
                for k in range(1, LANES):
                    cur_d = c[0]
                    d = dvec[k]
                    i = j * LANES + k
                    neq = d != cur_d

                    @pl.when(jnp.logical_and(neq, jnp.logical_and(
                        cur_d >= 0, cur_d < row_count)))
                    def _(cur_d=cur_d, avs=c[1:]):
                        for v in range(H // LANES):
                            sl = pl.ds(v * LANES, LANES)
                            acc[cur_d, sl] = jnp.maximum(acc[cur_d, sl],
                                                         avs[v])

                    na = []
                    for v in range(H // LANES):
                        ev = e_buf[par, i, pl.ds(v * LANES, LANES)]
                        na.append(jnp.where(neq, ev,
                                            jnp.maximum(c[1 + v], ev)))
                    c = (d,) + tuple(na)

                cur_d = c[0]

                @pl.when(jnp.logical_and(cur_d >= 0, cur_d < row_count))
                def _(cur_d=cur_d, avs=c[1:]):
                    for v in range(H // LANES):
                        sl = pl.ds(v * LANES, LANES)
                        acc[cur_d, sl] = jnp.maximum(acc[cur_d, sl], avs[v])

            return 0

        lax.fori_loop(0, WB // LANES, grp, 0)

    def pair(wp, _):
        for b in (0, 1):
            w = 2 * wp + b

            @pl.when(w + 1 < n_win)
            def _(w=w, nb=1 - b):
                for c in _copies(w + 1, nb):
                    c.start()

            @pl.when(w < n_win)
            def _(w=w, b=b):
                for c in _copies(w, b):
                    c.wait()

            par = jnp.where(w < n_win, b, 1 - b)
            _window(w, par)
        return 0

    lax.fori_loop(0, (n_win + 1) // 2, pair, 0)

    pltpu.sync_copy(acc, agg_hbm.at[pl.ds(base, ROWS_PER_TILE), :])


def _b1_gather_relu(st, ssrc, sdst):
    return _build_b1()(st, ssrc, sdst)


def _b3_segmax(e, sdst, bounds):
    return _build_b3()(e, sdst, bounds)


BLK = 1000
GRID = N // BLK
EBLK = 2000
EGRID = E // EBLK

_full = lambda shape: pl.BlockSpec(shape, lambda i: (0, 0))
_rows = lambda w: pl.BlockSpec((BLK, w), lambda i: (i, 0))


def _dot(a, b):
    return jnp.dot(a, b, preferred_element_type=jnp.float32)


def _st_tables(z, pos_blk, wh1, bh1, wh2, bh2, a3, w1x, bf1):
    p = _dot(pos_blk, a3)
    delta = _dot(jnp.maximum(_dot(z, wh1) + bh1, 0.0), wh2) + bh2
    t = _dot(delta, a3) - p
    s = _dot(z, w1x) + p + bf1
    return jnp.concatenate([s, t], axis=-1)


def _pre_body(x_ref, pos_ref, we1, be1, we2, be2, wt, bt,
              wh1, bh1, wh2, bh2, a3, w1x, bf1,
              z_ref, st_ref):
    h = jnp.maximum(_dot(x_ref[...], we1[...]) + be1[...], 0.0)
    z = _dot(_dot(h, we2[...]) + be2[...], wt[...]) + bt[...]
    z_ref[...] = z
    st_ref[...] = _st_tables(z, pos_ref[...], wh1[...], bh1[...], wh2[...],
                             bh2[...], a3[...], w1x[...], bf1[...])


def _gmlp(agg_ref, z, wg1, bg1, wg2, bg2, final_relu):
    a = agg_ref[...]
    a = jnp.where(jnp.isfinite(a), a, 0.0)
    u = jnp.maximum(_dot(a, wg1[...]) + bg1[...], 0.0)
    out = _dot(u, wg2[...]) + bg2[...] + z
    if final_relu:
        out = jnp.maximum(out, 0.0)
    return out


def _mid_body(agg_ref, z_ref, pos_ref, wg1, bg1, wg2, bg2,
              wh1, bh1, wh2, bh2, a3, w1x, bf1,
              z_out, st_ref):
    z = _gmlp(agg_ref, z_ref[...], wg1, bg1, wg2, bg2, True)
    z_out[...] = z
    st_ref[...] = _st_tables(z, pos_ref[...], wh1[...], bh1[...], wh2[...],
                             bh2[...], a3[...], w1x[...], bf1[...])


def _mid3_body(agg_ref, z_ref, pos_ref, wg1, bg1, wg2, bg2, wp, bp,
               wh1, bh1, wh2, bh2, a3, w1x, bf1,
               z_out, st_ref):
    z = _gmlp(agg_ref, z_ref[...], wg1, bg1, wg2, bg2, True)
    w = _dot(z, wp[...]) + bp[...]
    z_out[...] = w
    st_ref[...] = _st_tables(w, pos_ref[...], wh1[...], bh1[...], wh2[...],
                             bh2[...], a3[...], w1x[...], bf1[...])


def _post_body(agg_ref, z_ref, wg1, bg1, wg2, bg2, wd1, bd1, wd2, bd2,
               out_ref):
    y = _gmlp(agg_ref, z_ref[...], wg1, bg1, wg2, bg2, False)
    h = jnp.maximum(_dot(y, wd1[...]) + bd1[...], 0.0)
    out_ref[...] = _dot(h, wd2[...]) + bd2[...]


def _b2_body(hid_ref, w2, b2, e_ref):
    e_ref[...] = _dot(hid_ref[...], w2[...]) + b2[...]


def _nodes_out(k):
    return [jax.ShapeDtypeStruct((N, k), jnp.float32)]


_W64 = _full((H, H))
_B64 = _full((1, H))
_A3 = _full((3, H))
_W3 = _full((H, 3))
_B3S = _full((1, 3))

_ST_WSPECS = [_W64, _B64, _W3, _B3S, _A3, _W64, _B64]

_pre_call = pl.pallas_call(
    _pre_body,
    grid=(GRID,),
    in_specs=[_rows(D_IN), _rows(3),
              _full((D_IN, D_IN)), _full((1, D_IN)), _full((D_IN, H)), _B64,
              _W64, _B64] + _ST_WSPECS,
    out_specs=[_rows(H), _rows(2 * H)],
    out_shape=[jax.ShapeDtypeStruct((N, H), jnp.float32),
               jax.ShapeDtypeStruct((N, 2 * H), jnp.float32)],
)

_mid_call = pl.pallas_call(
    _mid_body,
    grid=(GRID,),
    in_specs=[_rows(H), _rows(H), _rows(3),
              _W64, _B64, _W64, _B64] + _ST_WSPECS,
    out_specs=[_rows(H), _rows(2 * H)],
    out_shape=[jax.ShapeDtypeStruct((N, H), jnp.float32),
               jax.ShapeDtypeStruct((N, 2 * H), jnp.float32)],
)

_mid3_call = pl.pallas_call(
    _mid3_body,
    grid=(GRID,),
    in_specs=[_rows(H), _rows(H), _rows(3),
              _W64, _B64, _W64, _B64, _W64, _B64] + _ST_WSPECS,
    out_specs=[_rows(H), _rows(2 * H)],
    out_shape=[jax.ShapeDtypeStruct((N, H), jnp.float32),
               jax.ShapeDtypeStruct((N, 2 * H), jnp.float32)],
)

_post_call = pl.pallas_call(
    _post_body,
    grid=(GRID,),
    in_specs=[_rows(H), _rows(H),
              _W64, _B64, _W64, _B64, _W64, _B64, _full((H, 4)), _full((1, 4))],
    out_specs=pl.BlockSpec((BLK, 4), lambda i: (i, 0)),
    out_shape=jax.ShapeDtypeStruct((N, 4), jnp.float32),
)

_b2_call = pl.pallas_call(
    _b2_body,
    grid=(EGRID,),
    in_specs=[pl.BlockSpec((EBLK, H), lambda i: (i, 0)), _W64, _B64],
    out_specs=pl.BlockSpec((EBLK, H), lambda i: (i, 0)),
    out_shape=jax.ShapeDtypeStruct((E, H), jnp.float32),
)


def _conv_weights(p):
    (wh1, bh1), (wh2, bh2) = p['h']
    (wf1, bf1), (wf2, bf2) = p['f']
    a3 = wf1[:3]
    w1x = wf1[3:]
    return (wh1, bh1.reshape(1, H), wh2, bh2.reshape(1, 3), a3, w1x,
            bf1.reshape(1, H)), (wf2, bf2.reshape(1, H))


def _g_weights(p):
    (wg1, bg1), (wg2, bg2) = p['g']
    return wg1, bg1.reshape(1, H), wg2, bg2.reshape(1, H)


def kernel(x, pos, edge_index, params):
    src = edge_index[0]
    dst = edge_index[1]
    sdst, ssrc = lax.sort((dst, src), num_keys=1)
    tile_edges = jnp.arange(0, (NW + 1) * ROWS_PER_TILE, ROWS_PER_TILE,
                            dtype=jnp.int32)
    cuts = jnp.searchsorted(sdst, tile_edges).astype(jnp.int32)
    se = jnp.stack([cuts[:-1], cuts[1:]], axis=1)
    bounds = jnp.pad(se, ((0, 0), (0, LANES - 2))).reshape(NW * LANES)

    (we1, be1), (we2, be2) = params['encoder']
    ((wt, bt),) = params['input_transform']

    st_w, f2_w = _conv_weights(params['convs'][0])
    z, st = _pre_call(x, pos, we1, be1.reshape(1, D_IN), we2,
                      be2.reshape(1, H), wt, bt.reshape(1, H), *st_w)

    convs = list(params['convs']) + [params['out_conv']]
    for k in range(4):
        wf2, bf2 = f2_w
        hid = _b1_gather_relu(st, ssrc, sdst)
        e = _b2_call(hid, wf2, bf2)
        aggp = _b3_segmax(e, sdst, bounds)
        agg = aggp[:N]
        gw = _g_weights(convs[k])
        if k < 2:
            st_w, f2_w = _conv_weights(convs[k + 1])
            z, st = _mid_call(agg, z, pos, *gw, *st_w)
        elif k == 2:
            ((wp, bp),) = params['out_projection']
            st_w, f2_w = _conv_weights(convs[3])
            z, st = _mid3_call(agg, z, pos, *gw, wp, bp.reshape(1, H),
                               *st_w)
        else:
            (wd1, bd1), (wd2, bd2) = params['decoder']
            out = _post_call(agg, z, *gw, wd1, bd1.reshape(1, H),
                             wd2, bd2.reshape(1, 4))
    return out


# speedup vs baseline: 3.0107x; 1.0189x over previous
import functools

import jax
import jax.numpy as jnp
from jax import lax
from jax.experimental import pallas as pl
from jax.experimental.pallas import tpu as pltpu
from jax.experimental.pallas import tpu_sc as plsc

N = 10000
E = 320000
D_IN = 128
H = 64

NC, NS, LANES = 2, 16, 16
NW = NC * NS
ROWS_PER_TILE = -(-N // (NW * 8)) * 8
N_PAD = NW * ROWS_PER_TILE
WB = 80
WB3 = 128
E_PER_W = E // NW
NWIN_B1 = E_PER_W // WB

NEG_INF = float("-inf")


def _mesh():
    return plsc.VectorSubcoreMesh(
        core_axis_name="c", subcore_axis_name="s",
        num_cores=NC, num_subcores=NS)


def _wid():
    return lax.axis_index("s") * NC + lax.axis_index("c")


@functools.cache
def _build_b1():
  return pl.kernel(
    _b1_body,
    out_type=jax.ShapeDtypeStruct((E, H), jnp.float32),
    mesh=_mesh(),
    scratch_types=[
        pltpu.VMEM((2, WB), jnp.int32),
        pltpu.VMEM((2, WB), jnp.int32),
        pltpu.VMEM((2, WB, 2 * H), jnp.float32),
        pltpu.VMEM((2, WB, 2 * H), jnp.float32),
        pltpu.VMEM((2, WB, H), jnp.float32),
        pltpu.SemaphoreType.DMA,
        pltpu.SemaphoreType.DMA,
        pltpu.SemaphoreType.DMA,
        pltpu.SemaphoreType.DMA,
        pltpu.SemaphoreType.DMA,
        pltpu.SemaphoreType.DMA,
    ],
  )


def _b1_body(st_hbm, src_hbm, dst_hbm, hid_hbm,
             idx_s, idx_t, rows_s, rows_t, hid_buf,
             si0, si1, sg0, sg1, sw0, sw1):
    base = _wid() * E_PER_W
    sem_i = (si0, si1)
    sem_g = (sg0, sg1)
    sem_w = (sw0, sw1)
    NWIN = NWIN_B1

    def idx_copies(w, b):
        ws = base + w * WB
        return (pltpu.make_async_copy(src_hbm.at[pl.ds(ws, WB)],
                                      idx_s.at[b], sem_i[b]),
                pltpu.make_async_copy(dst_hbm.at[pl.ds(ws, WB)],
                                      idx_t.at[b], sem_i[b]))

    def gather_copies(b):
        return (pltpu.make_async_copy(st_hbm.at[idx_s.at[b]],
                                      rows_s.at[b], sem_g[b]),
                pltpu.make_async_copy(st_hbm.at[idx_t.at[b]],
                                      rows_t.at[b], sem_g[b]))

    def wb_copy(w, b):
        ws = base + w * WB
        return pltpu.make_async_copy(hid_buf.at[b],
                                     hid_hbm.at[pl.ds(ws, WB), :], sem_w[b])

    for c in idx_copies(0, 0):
        c.start()
    for c in idx_copies(0, 0):
        c.wait()
    for c in gather_copies(0):
        c.start()
    for c in idx_copies(1, 1):
        c.start()

    def step(w, b):
        nb = 1 - b

        @pl.when(w + 1 < NWIN)
        def _():
            for c in idx_copies(w + 1, nb):
                c.wait()
            for c in gather_copies(nb):
                c.start()

        @pl.when(w < NWIN)
        def _():
            for c in gather_copies(b):
                c.wait()

        @pl.when(w + 2 < NWIN)
        def _():
            for c in idx_copies(w + 2, b):
                c.start()

        @pl.when(jnp.logical_and(w >= 2, w - 2 < NWIN))
        def _():
            wb_copy(w - 2, b).wait()

        for i in range(WB):
            for v in range(H // LANES):
                sl = pl.ds(v * LANES, LANES)
                tl = pl.ds(H + v * LANES, LANES)
                hid_buf[b, i, sl] = jnp.maximum(
                    rows_s[b, i, sl] + rows_t[b, i, tl], 0.0)

        @pl.when(w < NWIN)
        def _():
            wb_copy(w, b).start()

    def pair(wp, _):
        for b in (0, 1):
            step(2 * wp + b, b)
        return 0

    lax.fori_loop(0, (NWIN + 1) // 2, pair, 0)
    wb_copy(NWIN - 1, (NWIN - 1) % 2).wait()


@functools.cache
def _build_b3():
  return pl.kernel(
    _b3_body,
    out_type=jax.ShapeDtypeStruct((N_PAD, H), jnp.float32),
    mesh=_mesh(),
    scratch_types=[
        pltpu.VMEM((ROWS_PER_TILE, H), jnp.float32),
        pltpu.VMEM((2, WB3, H), jnp.float32),
        pltpu.VMEM((2, WB3), jnp.int32),
        pltpu.VMEM((NW * LANES,), jnp.int32),
        pltpu.SemaphoreType.DMA,
        pltpu.SemaphoreType.DMA,
        pltpu.SemaphoreType.DMA,
        pltpu.SemaphoreType.DMA,
    ],
  )


def _b3_body(e_hbm, dst_hbm, bounds_hbm, agg_hbm, acc, e_buf, dst_buf, bnd,
             se0, se1, sd0, sd1):
    wid = _wid()
    base = wid * ROWS_PER_TILE
    row_count = jnp.minimum(ROWS_PER_TILE, N - base)
    sem_e = (se0, se1)
    sem_d = (sd0, sd1)

    pltpu.sync_copy(bounds_hbm, bnd)

    def initrow(i, _):
        for v in range(H // LANES):
            acc[i, pl.ds(v * LANES, LANES)] = jnp.full((LANES,), NEG_INF,
                                                       jnp.float32)
        return 0

    lax.fori_loop(0, ROWS_PER_TILE, initrow, 0)

    bv = bnd[pl.ds(wid * LANES, LANES)]
    start = bv[0]
    end = bv[1]
    s8 = start // 8
    astart = s8 * 8
    cap8 = (E - WB3) // 8
    n_win = jnp.maximum((end - astart + WB3 - 1) // WB3, 0)

    def _ws(w):
        return jnp.minimum(s8 + w * (WB3 // 8), cap8) * 8

    def _copies(w, b):
        ws = _ws(w)
        return (pltpu.make_async_copy(e_hbm.at[pl.ds(ws, WB3), :],
                                      e_buf.at[b], sem_e[b]),
                pltpu.make_async_copy(dst_hbm.at[pl.ds(ws, WB3)],
                                      dst_buf.at[b], sem_d[b]))

    @pl.when(n_win > 0)
    def _():
        for c in _copies(0, 0):
            c.start()

    def _window(w, par):
        def grp(j, _):
            dvec = dst_buf[par, pl.ds(j * LANES, LANES)] - base
            d0 = dvec[0]
            dlast = dvec[LANES - 1]
            uniform = d0 == dlast
            valid0 = jnp.logical_and(d0 >= 0, d0 < row_count)

            @pl.when(jnp.logical_and(uniform, valid0))
            def _():
                for v in range(H // LANES):
                    sl = pl.ds(v * LANES, LANES)
                    m = e_buf[par, j * LANES, sl]
                    for k in range(1, LANES):
                        m = jnp.maximum(m, e_buf[par, j * LANES + k, sl])
                    acc[d0, sl] = jnp.maximum(acc[d0, sl], m)

            @pl.when(jnp.logical_not(uniform))
            def _():
                c = (d0,) + tuple(
                    e_buf[par, j * LANES, pl.ds(v * LANES, LANES)]
                    for v in range(H // LANES))
                for k in range(1, LANES):
                    cur_d = c[0]
                    d = dvec[k]
                    i = j * LANES + k
                    neq = d != cur_d

                    @pl.when(jnp.logical_and(neq, jnp.logical_and(
                        cur_d >= 0, cur_d < row_count)))
                    def _(cur_d=cur_d, avs=c[1:]):
                        for v in range(H // LANES):
                            sl = pl.ds(v * LANES, LANES)
                            acc[cur_d, sl] = jnp.maximum(acc[cur_d, sl],
                                                         avs[v])

                    na = []
                    for v in range(H // LANES):
                        ev = e_buf[par, i, pl.ds(v * LANES, LANES)]
                        na.append(jnp.where(neq, ev,
                                            jnp.maximum(c[1 + v], ev)))
                    c = (d,) + tuple(na)

                cur_d = c[0]

                @pl.when(jnp.logical_and(cur_d >= 0, cur_d < row_count))
                def _(cur_d=cur_d, avs=c[1:]):
                    for v in range(H // LANES):
                        sl = pl.ds(v * LANES, LANES)
                        acc[cur_d, sl] = jnp.maximum(acc[cur_d, sl], avs[v])

            return 0

        lax.fori_loop(0, WB3 // LANES, grp, 0)

    def pair(wp, _):
        for b in (0, 1):
            w = 2 * wp + b

            @pl.when(w + 1 < n_win)
            def _(w=w, nb=1 - b):
                for c in _copies(w + 1, nb):
                    c.start()

            @pl.when(w < n_win)
            def _(w=w, b=b):
                for c in _copies(w, b):
                    c.wait()

            par = jnp.where(w < n_win, b, 1 - b)
            _window(w, par)
        return 0

    lax.fori_loop(0, (n_win + 1) // 2, pair, 0)

    pltpu.sync_copy(acc, agg_hbm.at[pl.ds(base, ROWS_PER_TILE), :])


def _b1_gather_relu(st, ssrc, sdst):
    return _build_b1()(st, ssrc, sdst)


def _b3_segmax(e, sdst, bounds):
    return _build_b3()(e, sdst, bounds)


BLK = 1000
GRID = N // BLK
EBLK = 2000
EGRID = E // EBLK

_full = lambda shape: pl.BlockSpec(shape, lambda i: (0, 0))
_rows = lambda w: pl.BlockSpec((BLK, w), lambda i: (i, 0))


def _dot(a, b):
    return jnp.dot(a, b, preferred_element_type=jnp.float32)


def _st_tables(z, pos_blk, wh1, bh1, wh2, bh2, a3, w1x, bf1):
    p = _dot(pos_blk, a3)
    delta = _dot(jnp.maximum(_dot(z, wh1) + bh1, 0.0), wh2) + bh2
    t = _dot(delta, a3) - p
    s = _dot(z, w1x) + p + bf1
    return jnp.concatenate([s, t], axis=-1)


def _pre_body(x_ref, pos_ref, we1, be1, we2, be2, wt, bt,
              wh1, bh1, wh2, bh2, a3, w1x, bf1,
              z_ref, st_ref):
    h = jnp.maximum(_dot(x_ref[...], we1[...]) + be1[...], 0.0)
    z = _dot(_dot(h, we2[...]) + be2[...], wt[...]) + bt[...]
    z_ref[...] = z
    st_ref[...] = _st_tables(z, pos_ref[...], wh1[...], bh1[...], wh2[...],
                             bh2[...], a3[...], w1x[...], bf1[...])


def _gmlp(agg_ref, z, wg1, bg1, wg2, bg2, final_relu):
    a = agg_ref[...]
    a = jnp.where(jnp.isfinite(a), a, 0.0)
    u = jnp.maximum(_dot(a, wg1[...]) + bg1[...], 0.0)
    out = _dot(u, wg2[...]) + bg2[...] + z
    if final_relu:
        out = jnp.maximum(out, 0.0)
    return out


def _mid_body(agg_ref, z_ref, pos_ref, wg1, bg1, wg2, bg2,
              wh1, bh1, wh2, bh2, a3, w1x, bf1,
              z_out, st_ref):
    z = _gmlp(agg_ref, z_ref[...], wg1, bg1, wg2, bg2, True)
    z_out[...] = z
    st_ref[...] = _st_tables(z, pos_ref[...], wh1[...], bh1[...], wh2[...],
                             bh2[...], a3[...], w1x[...], bf1[...])


def _mid3_body(agg_ref, z_ref, pos_ref, wg1, bg1, wg2, bg2, wp, bp,
               wh1, bh1, wh2, bh2, a3, w1x, bf1,
               z_out, st_ref):
    z = _gmlp(agg_ref, z_ref[...], wg1, bg1, wg2, bg2, True)
    w = _dot(z, wp[...]) + bp[...]
    z_out[...] = w
    st_ref[...] = _st_tables(w, pos_ref[...], wh1[...], bh1[...], wh2[...],
                             bh2[...], a3[...], w1x[...], bf1[...])


def _post_body(agg_ref, z_ref, wg1, bg1, wg2, bg2, wd1, bd1, wd2, bd2,
               out_ref):
    y = _gmlp(agg_ref, z_ref[...], wg1, bg1, wg2, bg2, False)
    h = jnp.maximum(_dot(y, wd1[...]) + bd1[...], 0.0)
    out_ref[...] = _dot(h, wd2[...]) + bd2[...]


def _b2_body(hid_ref, w2, b2, e_ref):
    e_ref[...] = _dot(hid_ref[...], w2[...]) + b2[...]


def _nodes_out(k):
    return [jax.ShapeDtypeStruct((N, k), jnp.float32)]


_W64 = _full((H, H))
_B64 = _full((1, H))
_A3 = _full((3, H))
_W3 = _full((H, 3))
_B3S = _full((1, 3))

_ST_WSPECS = [_W64, _B64, _W3, _B3S, _A3, _W64, _B64]

_pre_call = pl.pallas_call(
    _pre_body,
    grid=(GRID,),
    in_specs=[_rows(D_IN), _rows(3),
              _full((D_IN, D_IN)), _full((1, D_IN)), _full((D_IN, H)), _B64,
              _W64, _B64] + _ST_WSPECS,
    out_specs=[_rows(H), _rows(2 * H)],
    out_shape=[jax.ShapeDtypeStruct((N, H), jnp.float32),
               jax.ShapeDtypeStruct((N, 2 * H), jnp.float32)],
)

_mid_call = pl.pallas_call(
    _mid_body,
    grid=(GRID,),
    in_specs=[_rows(H), _rows(H), _rows(3),
              _W64, _B64, _W64, _B64] + _ST_WSPECS,
    out_specs=[_rows(H), _rows(2 * H)],
    out_shape=[jax.ShapeDtypeStruct((N, H), jnp.float32),
               jax.ShapeDtypeStruct((N, 2 * H), jnp.float32)],
)

_mid3_call = pl.pallas_call(
    _mid3_body,
    grid=(GRID,),
    in_specs=[_rows(H), _rows(H), _rows(3),
              _W64, _B64, _W64, _B64, _W64, _B64] + _ST_WSPECS,
    out_specs=[_rows(H), _rows(2 * H)],
    out_shape=[jax.ShapeDtypeStruct((N, H), jnp.float32),
               jax.ShapeDtypeStruct((N, 2 * H), jnp.float32)],
)

_post_call = pl.pallas_call(
    _post_body,
    grid=(GRID,),
    in_specs=[_rows(H), _rows(H),
              _W64, _B64, _W64, _B64, _W64, _B64, _full((H, 4)), _full((1, 4))],
    out_specs=pl.BlockSpec((BLK, 4), lambda i: (i, 0)),
    out_shape=jax.ShapeDtypeStruct((N, 4), jnp.float32),
)

_b2_call = pl.pallas_call(
    _b2_body,
    grid=(EGRID,),
    in_specs=[pl.BlockSpec((EBLK, H), lambda i: (i, 0)), _W64, _B64],
    out_specs=pl.BlockSpec((EBLK, H), lambda i: (i, 0)),
    out_shape=jax.ShapeDtypeStruct((E, H), jnp.float32),
)


def _conv_weights(p):
    (wh1, bh1), (wh2, bh2) = p['h']
    (wf1, bf1), (wf2, bf2) = p['f']
    a3 = wf1[:3]
    w1x = wf1[3:]
    return (wh1, bh1.reshape(1, H), wh2, bh2.reshape(1, 3), a3, w1x,
            bf1.reshape(1, H)), (wf2, bf2.reshape(1, H))


def _g_weights(p):
    (wg1, bg1), (wg2, bg2) = p['g']
    return wg1, bg1.reshape(1, H), wg2, bg2.reshape(1, H)


def kernel(x, pos, edge_index, params):
    src = edge_index[0]
    dst = edge_index[1]
    sdst, ssrc = lax.sort((dst, src), num_keys=1)
    tile_edges = jnp.arange(0, (NW + 1) * ROWS_PER_TILE, ROWS_PER_TILE,
                            dtype=jnp.int32)
    cuts = jnp.searchsorted(sdst, tile_edges).astype(jnp.int32)
    se = jnp.stack([cuts[:-1], cuts[1:]], axis=1)
    bounds = jnp.pad(se, ((0, 0), (0, LANES - 2))).reshape(NW * LANES)

    (we1, be1), (we2, be2) = params['encoder']
    ((wt, bt),) = params['input_transform']

    st_w, f2_w = _conv_weights(params['convs'][0])
    z, st = _pre_call(x, pos, we1, be1.reshape(1, D_IN), we2,
                      be2.reshape(1, H), wt, bt.reshape(1, H), *st_w)

    convs = list(params['convs']) + [params['out_conv']]
    for k in range(4):
        wf2, bf2 = f2_w
        hid = _b1_gather_relu(st, ssrc, sdst)
        e = _b2_call(hid, wf2, bf2)
        aggp = _b3_segmax(e, sdst, bounds)
        agg = aggp[:N]
        gw = _g_weights(convs[k])
        if k < 2:
            st_w, f2_w = _conv_weights(convs[k + 1])
            z, st = _mid_call(agg, z, pos, *gw, *st_w)
        elif k == 2:
            ((wp, bp),) = params['out_projection']
            st_w, f2_w = _conv_weights(convs[3])
            z, st = _mid3_call(agg, z, pos, *gw, wp, bp.reshape(1, H),
                               *st_w)
        else:
            (wd1, bd1), (wd2, bd2) = params['decoder']
            out = _post_call(agg, z, *gw, wd1, bd1.reshape(1, H),
                             wd2, bd2.reshape(1, 4))
    return out
